# Initial kernel scaffold; baseline (speedup 1.0000x reference)
import jax
import jax.numpy as jnp
from jax.experimental import pallas as pl


def kernel(ms, weights, ray_indices, num_rays):
    raise NotImplementedError("write your pallas kernel here")


# speedup vs baseline: 5.2803x; 5.2803x over previous
import functools

import jax
import jax.numpy as jnp
from jax import lax
from jax.experimental import pallas as pl
from jax.experimental.pallas import tpu as pltpu
from jax.experimental.pallas import tpu_sc as plsc

N_SAMPLES = 3145728
N_CH = 8
N_RAYS = 65536
NC = 2
NS = 16
NW = NC * NS
CHUNK = N_SAMPLES // NW
BLK = 2048
NBLK = CHUNK // BLK
ROWS_PER_SUB = N_RAYS // NS


def _sc_segment_sum(ms_pair, w, ridx):
  mesh = plsc.VectorSubcoreMesh(core_axis_name="c", subcore_axis_name="s")

  @functools.partial(
      pl.kernel,
      out_type=jax.ShapeDtypeStruct((NC, N_RAYS, N_CH), jnp.float32),
      mesh=mesh,
      scratch_types=dict(
          acc=pltpu.VMEM_SHARED((N_RAYS, N_CH), jnp.float32),
          ms_v=pltpu.VMEM((BLK // 2, 16), jnp.float32),
          w_v=pltpu.VMEM((BLK,), jnp.float32),
          idx_v=pltpu.VMEM((BLK,), jnp.int32),
          vals_v=pltpu.VMEM((BLK, N_CH), jnp.float32),
      ),
      compiler_params=pltpu.CompilerParams(use_tc_tiling_on_sc=False,
                                           needs_layout_passes=False),
  )
  def seg_sum(ms_hbm, w_hbm, idx_hbm, out_hbm, *, acc, ms_v, w_v, idx_v,
              vals_v):
    cid = lax.axis_index("c")
    sid = lax.axis_index("s")
    wid = cid * NS + sid

    iota = lax.iota(jnp.int32, 16)
    hi = iota >> 3
    lo = iota & 7

    zeros16 = jnp.zeros((16,), jnp.float32)

    def zero_body(i, _):
      plsc.store_scatter(vals_v, [2 * i + hi, lo], zeros16)
      return 0

    lax.fori_loop(0, BLK // 2, zero_body, 0)
    for rep in range(ROWS_PER_SUB // BLK):
      row0 = sid * ROWS_PER_SUB + rep * BLK
      pltpu.sync_copy(vals_v, acc.at[pl.ds(row0, BLK)])
    plsc.subcore_barrier()

    def block_body(b, _):
      s0 = pl.multiple_of(wid * CHUNK + b * BLK, 1024)
      pltpu.sync_copy(ms_hbm.at[pl.ds(pl.multiple_of(s0 // 2, 512), BLK // 2)],
                      ms_v)
      pltpu.sync_copy(w_hbm.at[pl.ds(s0, BLK)], w_v)
      pltpu.sync_copy(idx_hbm.at[pl.ds(s0, BLK)], idx_v)

      def pair_body(i, _):
        iv = 2 * i + hi
        m16 = ms_v[i]
        w16 = plsc.load_gather(w_v, [iv])
        plsc.store_scatter(vals_v, [iv, lo], m16 * w16)
        return 0

      lax.fori_loop(0, BLK // 2, pair_body, 0)
      pltpu.sync_copy(vals_v, acc.at[idx_v], add=True)
      return 0

    lax.fori_loop(0, NBLK, block_body, 0)
    plsc.subcore_barrier()

    row0 = sid * ROWS_PER_SUB
    pltpu.sync_copy(acc.at[pl.ds(row0, ROWS_PER_SUB)],
                    out_hbm.at[cid, pl.ds(row0, ROWS_PER_SUB)])

  return seg_sum(ms_pair, w, ridx)


def _tc_combine(partials):
  flat = partials.reshape(NC, N_RAYS * N_CH // 128, 128)

  def add_body(a_ref, b_ref, o_ref):
    o_ref[...] = a_ref[...] + b_ref[...]

  out = pl.pallas_call(
      add_body,
      out_shape=jax.ShapeDtypeStruct((N_RAYS * N_CH // 128, 128), jnp.float32),
      in_specs=[
          pl.BlockSpec((N_RAYS * N_CH // 128, 128), lambda: (0, 0)),
          pl.BlockSpec((N_RAYS * N_CH // 128, 128), lambda: (0, 0)),
      ],
      out_specs=pl.BlockSpec((N_RAYS * N_CH // 128, 128), lambda: (0, 0)),
  )(flat[0], flat[1])
  return out.reshape(N_RAYS, N_CH)


def kernel(ms, weights, ray_indices, num_rays):
  del num_rays
  ms_pair = ms.reshape(N_SAMPLES // 2, 16)
  w = weights.reshape(N_SAMPLES)
  ridx = ray_indices.astype(jnp.int32)
  partials = _sc_segment_sum(ms_pair, w, ridx)
  return _tc_combine(partials)


# speedup vs baseline: 20.1701x; 3.8199x over previous
import functools

import jax
import jax.numpy as jnp
from jax import lax
from jax.experimental import pallas as pl
from jax.experimental.pallas import tpu as pltpu
from jax.experimental.pallas import tpu_sc as plsc

N_SAMPLES = 3145728
N_CH = 8
N_RAYS = 65536
NC = 2
NS = 16
NW = NC * NS
CHUNK = N_SAMPLES // NW
BLK = 2048
NBLK = CHUNK // BLK
GRP = BLK // 16
MS_ROWS = BLK // 128 * N_CH
ROWS_PER_SUB = N_RAYS // NS


def _sc_segment_sum(ms_lin, w, ridx):
  mesh = plsc.VectorSubcoreMesh(core_axis_name="c", subcore_axis_name="s")

  @functools.partial(
      pl.kernel,
      out_type=jax.ShapeDtypeStruct((NC, N_RAYS, N_CH), jnp.float32),
      mesh=mesh,
      scratch_types=dict(
          acc=pltpu.VMEM_SHARED((N_RAYS, N_CH), jnp.float32),
          ms_v=pltpu.VMEM((MS_ROWS, 128), jnp.float32),
          w_v=pltpu.VMEM((BLK,), jnp.float32),
          idx_v=pltpu.VMEM((BLK,), jnp.int32),
          vals_v=pltpu.VMEM((BLK, N_CH), jnp.float32),
      ),
      compiler_params=pltpu.CompilerParams(use_tc_tiling_on_sc=False,
                                           needs_layout_passes=False),
  )
  def seg_sum(ms_hbm, w_hbm, idx_hbm, out_hbm, *, acc, ms_v, w_v, idx_v,
              vals_v):
    cid = lax.axis_index("c")
    sid = lax.axis_index("s")
    wid = cid * NS + sid

    iota = lax.iota(jnp.int32, 16)
    hi = iota >> 3
    lo = iota & 7
    zeros16 = jnp.zeros((16,), jnp.float32)

    def zero_body(i, _):
      plsc.store_scatter(vals_v, [2 * i + hi, lo], zeros16)
      return 0

    lax.fori_loop(0, BLK // 2, zero_body, 0)
    for rep in range(ROWS_PER_SUB // BLK):
      row0 = sid * ROWS_PER_SUB + rep * BLK
      pltpu.sync_copy(vals_v, acc.at[pl.ds(row0, BLK)])
    plsc.subcore_barrier()

    def block_body(b, _):
      s0 = pl.multiple_of(wid * CHUNK + b * BLK, BLK)
      r0 = pl.multiple_of(s0 // 16, MS_ROWS)
      pltpu.sync_copy(ms_hbm.at[pl.ds(r0, MS_ROWS)], ms_v)
      pltpu.sync_copy(w_hbm.at[pl.ds(s0, BLK)], w_v)
      pltpu.sync_copy(idx_hbm.at[pl.ds(s0, BLK)], idx_v)

      def group_body(g, _):
        col = pl.multiple_of((g % (128 // 16)) * 16, 16)
        rowb = (g // (128 // 16)) * N_CH
        samp = 16 * g + iota
        w16 = w_v[pl.ds(pl.multiple_of(16 * g, 16), 16)]
        for c in range(N_CH):
          m16 = ms_v[rowb + c, pl.ds(col, 16)]
          plsc.store_scatter(vals_v,
                             [samp, jnp.full((16,), c, jnp.int32)],
                             m16 * w16)
        return 0

      lax.fori_loop(0, GRP, group_body, 0)
      pltpu.sync_copy(vals_v, acc.at[idx_v], add=True)
      return 0

    lax.fori_loop(0, NBLK, block_body, 0)
    plsc.subcore_barrier()

    row0 = sid * ROWS_PER_SUB
    pltpu.sync_copy(acc.at[pl.ds(row0, ROWS_PER_SUB)],
                    out_hbm.at[cid, pl.ds(row0, ROWS_PER_SUB)])

  return seg_sum(ms_lin, w, ridx)


def _tc_combine(partials):
  flat = partials.reshape(NC, N_RAYS * N_CH // 128, 128)
  rows = N_RAYS * N_CH // 128

  def add_body(p_ref, o_ref):
    o_ref[...] = p_ref[0] + p_ref[1]

  out = pl.pallas_call(
      add_body,
      out_shape=jax.ShapeDtypeStruct((rows, 128), jnp.float32),
      in_specs=[pl.BlockSpec((NC, rows, 128), lambda: (0, 0, 0))],
      out_specs=pl.BlockSpec((rows, 128), lambda: (0, 0)),
  )(flat)
  return out.reshape(N_RAYS, N_CH)


def kernel(ms, weights, ray_indices, num_rays):
  del num_rays
  ms_lin = (ms.reshape(N_SAMPLES // 128, 128, N_CH)
            .transpose(0, 2, 1)
            .reshape(N_SAMPLES // 128 * N_CH, 128))
  w = weights.reshape(N_SAMPLES)
  ridx = ray_indices.astype(jnp.int32)
  partials = _sc_segment_sum(ms_lin, w, ridx)
  return _tc_combine(partials)


# speedup vs baseline: 27.5120x; 1.3640x over previous
import functools

import jax
import jax.numpy as jnp
from jax import lax
from jax.experimental import pallas as pl
from jax.experimental.pallas import tpu as pltpu
from jax.experimental.pallas import tpu_sc as plsc

N_SAMPLES = 3145728
N_CH = 8
N_RAYS = 65536
NC = 2
NS = 16
NW = NC * NS
CHUNK = N_SAMPLES // NW
BLK = 2048
NBLK = CHUNK // BLK
GRP = BLK // 16
MS_ROWS = BLK // 128 * N_CH
ROWS_PER_SUB = N_RAYS // NS
NBUF = 2


def _sc_segment_sum(ms_lin, w, ridx):
  mesh = plsc.VectorSubcoreMesh(core_axis_name="c", subcore_axis_name="s")

  @functools.partial(
      pl.kernel,
      out_type=jax.ShapeDtypeStruct((NC, N_RAYS, N_CH), jnp.float32),
      mesh=mesh,
      scratch_types=dict(
          acc=pltpu.VMEM_SHARED((N_RAYS, N_CH), jnp.float32),
          ms_v=tuple(pltpu.VMEM((MS_ROWS, 128), jnp.float32)
                     for _ in range(NBUF)),
          w_v=tuple(pltpu.VMEM((BLK,), jnp.float32) for _ in range(NBUF)),
          idx_v=tuple(pltpu.VMEM((BLK,), jnp.int32) for _ in range(NBUF)),
          vals_v=tuple(pltpu.VMEM((BLK, N_CH), jnp.float32)
                       for _ in range(NBUF)),
          in_sems=tuple(pltpu.SemaphoreType.DMA((3,)) for _ in range(NBUF)),
          sc_sems=tuple(pltpu.SemaphoreType.DMA for _ in range(NBUF)),
      ),
      compiler_params=pltpu.CompilerParams(use_tc_tiling_on_sc=False,
                                           needs_layout_passes=False),
  )
  def seg_sum(ms_hbm, w_hbm, idx_hbm, out_hbm, *, acc, ms_v, w_v, idx_v,
              vals_v, in_sems, sc_sems):
    cid = lax.axis_index("c")
    sid = lax.axis_index("s")
    wid = cid * NS + sid

    iota = lax.iota(jnp.int32, 16)
    hi = iota >> 3
    lo = iota & 7
    zeros16 = jnp.zeros((16,), jnp.float32)

    def zero_body(i, _):
      plsc.store_scatter(vals_v[0], [2 * i + hi, lo], zeros16)
      return 0

    lax.fori_loop(0, BLK // 2, zero_body, 0)
    for rep in range(ROWS_PER_SUB // BLK):
      row0 = sid * ROWS_PER_SUB + rep * BLK
      pltpu.sync_copy(vals_v[0], acc.at[pl.ds(row0, BLK)])
    plsc.subcore_barrier()

    def start_in(b, k):
      s0 = pl.multiple_of(wid * CHUNK + b * BLK, BLK)
      r0 = pl.multiple_of(s0 // 16, MS_ROWS)
      pltpu.async_copy(ms_hbm.at[pl.ds(r0, MS_ROWS)], ms_v[k],
                       in_sems[k].at[0])
      pltpu.async_copy(w_hbm.at[pl.ds(s0, BLK)], w_v[k], in_sems[k].at[1])
      pltpu.async_copy(idx_hbm.at[pl.ds(s0, BLK)], idx_v[k],
                       in_sems[k].at[2])

    def wait_in(b, k):
      s0 = pl.multiple_of(wid * CHUNK + b * BLK, BLK)
      r0 = pl.multiple_of(s0 // 16, MS_ROWS)
      pltpu.make_async_copy(ms_hbm.at[pl.ds(r0, MS_ROWS)], ms_v[k],
                            in_sems[k].at[0]).wait()
      pltpu.make_async_copy(w_hbm.at[pl.ds(s0, BLK)], w_v[k],
                            in_sems[k].at[1]).wait()
      pltpu.make_async_copy(idx_hbm.at[pl.ds(s0, BLK)], idx_v[k],
                            in_sems[k].at[2]).wait()

    def wait_scatter(k):
      pltpu.make_async_copy(vals_v[k], acc.at[idx_v[k]], sc_sems[k]).wait()

    def compute_block(b, k):
      def group_body(g, _):
        col = pl.multiple_of((g % (128 // 16)) * 16, 16)
        rowb = (g // (128 // 16)) * N_CH
        samp = 16 * g + iota
        w16 = w_v[k][pl.ds(pl.multiple_of(16 * g, 16), 16)]
        for c in range(N_CH):
          m16 = ms_v[k][rowb + c, pl.ds(col, 16)]
          plsc.store_scatter(vals_v[k],
                             [samp, jnp.full((16,), c, jnp.int32)],
                             m16 * w16)
        return 0

      lax.fori_loop(0, GRP, group_body, 0, unroll=2)

    start_in(0, 0)

    def super_body(sb, _):
      for k in range(NBUF):
        b = sb * NBUF + k
        nk = (k + 1) % NBUF
        @pl.when(b >= 1)
        def _():
          wait_scatter(nk)

        @pl.when(b + 1 < NBLK)
        def _():
          start_in(b + 1, nk)

        wait_in(b, k)
        compute_block(b, k)
        pltpu.async_copy(vals_v[k], acc.at[idx_v[k]], sc_sems[k], add=True)
      return 0

    lax.fori_loop(0, NBLK // NBUF, super_body, 0)
    wait_scatter((NBLK - 1) % NBUF)
    plsc.subcore_barrier()

    row0 = sid * ROWS_PER_SUB
    pltpu.sync_copy(acc.at[pl.ds(row0, ROWS_PER_SUB)],
                    out_hbm.at[cid, pl.ds(row0, ROWS_PER_SUB)])

  return seg_sum(ms_lin, w, ridx)


def _tc_combine(partials):
  flat = partials.reshape(NC, N_RAYS * N_CH // 128, 128)
  rows = N_RAYS * N_CH // 128

  def add_body(p_ref, o_ref):
    o_ref[...] = p_ref[0] + p_ref[1]

  out = pl.pallas_call(
      add_body,
      out_shape=jax.ShapeDtypeStruct((rows, 128), jnp.float32),
      in_specs=[pl.BlockSpec((NC, rows, 128), lambda: (0, 0, 0))],
      out_specs=pl.BlockSpec((rows, 128), lambda: (0, 0)),
  )(flat)
  return out.reshape(N_RAYS, N_CH)


def kernel(ms, weights, ray_indices, num_rays):
  del num_rays
  ms_lin = (ms.reshape(N_SAMPLES // 128, 128, N_CH)
            .transpose(0, 2, 1)
            .reshape(N_SAMPLES // 128 * N_CH, 128))
  w = weights.reshape(N_SAMPLES)
  ridx = ray_indices.astype(jnp.int32)
  partials = _sc_segment_sum(ms_lin, w, ridx)
  return _tc_combine(partials)


# speedup vs baseline: 44.1509x; 1.6048x over previous
import functools

import jax
import jax.numpy as jnp
from jax import lax
from jax.experimental import pallas as pl
from jax.experimental.pallas import tpu as pltpu
from jax.experimental.pallas import tpu_sc as plsc

N_SAMPLES = 3145728
N_CH = 8
N_RAYS = 65536
NC = 2
NS = 16
NW = NC * NS
CHUNK = N_SAMPLES // NW
BLK = 2048
NBLK = CHUNK // BLK
GRP = BLK // 16
MS_ROWS = BLK // 128 * N_CH
ROWS_PER_SUB = N_RAYS // NS
NBUF = 2


def _sc_segment_sum(ms_lin, w, ridx):
  mesh = plsc.VectorSubcoreMesh(core_axis_name="c", subcore_axis_name="s")

  @functools.partial(
      pl.kernel,
      out_type=jax.ShapeDtypeStruct((NC, N_RAYS, N_CH), jnp.float32),
      mesh=mesh,
      scratch_types=dict(
          acc=pltpu.VMEM_SHARED((N_RAYS, N_CH), jnp.float32),
          ms_v=tuple(pltpu.VMEM((MS_ROWS, 128), jnp.float32)
                     for _ in range(NBUF)),
          w_v=tuple(pltpu.VMEM((BLK,), jnp.float32) for _ in range(NBUF)),
          idx_v=tuple(pltpu.VMEM((BLK,), jnp.int32) for _ in range(NBUF)),
          vals_v=tuple(pltpu.VMEM((BLK, N_CH), jnp.float32)
                       for _ in range(NBUF)),
          in_sems=tuple(pltpu.SemaphoreType.DMA((3,)) for _ in range(NBUF)),
          sc_sems=tuple(pltpu.SemaphoreType.DMA for _ in range(NBUF)),
      ),
      compiler_params=pltpu.CompilerParams(use_tc_tiling_on_sc=False,
                                           needs_layout_passes=False),
  )
  def seg_sum(ms_hbm, w_hbm, idx_hbm, out_hbm, *, acc, ms_v, w_v, idx_v,
              vals_v, in_sems, sc_sems):
    cid = lax.axis_index("c")
    sid = lax.axis_index("s")
    wid = cid * NS + sid

    iota = lax.iota(jnp.int32, 16)
    hi = iota >> 3
    lo = iota & 7
    zeros16 = jnp.zeros((16,), jnp.float32)

    def zero_body(i, _):
      plsc.store_scatter(vals_v[0], [2 * i + hi, lo], zeros16)
      return 0

    lax.fori_loop(0, BLK // 2, zero_body, 0)
    for rep in range(ROWS_PER_SUB // BLK):
      row0 = sid * ROWS_PER_SUB + rep * BLK
      pltpu.sync_copy(vals_v[0], acc.at[pl.ds(row0, BLK)])
    plsc.subcore_barrier()

    def start_in(b, k):
      s0 = pl.multiple_of(wid * CHUNK + b * BLK, BLK)
      r0 = pl.multiple_of(s0 // 16, MS_ROWS)
      pltpu.async_copy(ms_hbm.at[pl.ds(r0, MS_ROWS)], ms_v[k],
                       in_sems[k].at[0])
      pltpu.async_copy(w_hbm.at[pl.ds(s0, BLK)], w_v[k], in_sems[k].at[1])
      pltpu.async_copy(idx_hbm.at[pl.ds(s0, BLK)], idx_v[k],
                       in_sems[k].at[2])

    def wait_in(b, k):
      s0 = pl.multiple_of(wid * CHUNK + b * BLK, BLK)
      r0 = pl.multiple_of(s0 // 16, MS_ROWS)
      pltpu.make_async_copy(ms_hbm.at[pl.ds(r0, MS_ROWS)], ms_v[k],
                            in_sems[k].at[0]).wait()
      pltpu.make_async_copy(w_hbm.at[pl.ds(s0, BLK)], w_v[k],
                            in_sems[k].at[1]).wait()
      pltpu.make_async_copy(idx_hbm.at[pl.ds(s0, BLK)], idx_v[k],
                            in_sems[k].at[2]).wait()

    def wait_scatter(k):
      pltpu.make_async_copy(vals_v[k], acc.at[idx_v[k]], sc_sems[k]).wait()

    def compute_block(b, k):
      def group_body(g, _):
        col = pl.multiple_of((g % (128 // 16)) * 16, 16)
        rowb = (g // (128 // 16)) * N_CH
        samp = 16 * g + iota
        w16 = w_v[k][pl.ds(pl.multiple_of(16 * g, 16), 16)]
        loads = [ms_v[k][rowb + c, pl.ds(col, 16)] for c in range(N_CH)]
        prods = [m * w16 for m in loads]
        for c in range(N_CH):
          plsc.store_scatter(vals_v[k],
                             [samp, jnp.full((16,), c, jnp.int32)],
                             prods[c])
        return 0

      lax.fori_loop(0, GRP, group_body, 0, unroll=2)

    start_in(0, 0)

    def super_body(sb, _):
      for k in range(NBUF):
        b = sb * NBUF + k
        nk = (k + 1) % NBUF
        @pl.when(b >= 1)
        def _():
          wait_scatter(nk)

        @pl.when(b + 1 < NBLK)
        def _():
          start_in(b + 1, nk)

        wait_in(b, k)
        compute_block(b, k)
        pltpu.async_copy(vals_v[k], acc.at[idx_v[k]], sc_sems[k], add=True)
      return 0

    lax.fori_loop(0, NBLK // NBUF, super_body, 0)
    wait_scatter((NBLK - 1) % NBUF)
    plsc.subcore_barrier()

    row0 = sid * ROWS_PER_SUB
    pltpu.sync_copy(acc.at[pl.ds(row0, ROWS_PER_SUB)],
                    out_hbm.at[cid, pl.ds(row0, ROWS_PER_SUB)])

  return seg_sum(ms_lin, w, ridx)


def _tc_combine(partials):
  flat = partials.reshape(NC, N_RAYS * N_CH // 128, 128)
  rows = N_RAYS * N_CH // 128

  def add_body(p_ref, o_ref):
    o_ref[...] = p_ref[0] + p_ref[1]

  out = pl.pallas_call(
      add_body,
      out_shape=jax.ShapeDtypeStruct((rows, 128), jnp.float32),
      in_specs=[pl.BlockSpec((NC, rows, 128), lambda: (0, 0, 0))],
      out_specs=pl.BlockSpec((rows, 128), lambda: (0, 0)),
  )(flat)
  return out.reshape(N_RAYS, N_CH)


def kernel(ms, weights, ray_indices, num_rays):
  del num_rays
  ms_lin = (ms.reshape(N_SAMPLES // 128, 128, N_CH)
            .transpose(0, 2, 1)
            .reshape(N_SAMPLES // 128 * N_CH, 128))
  w = weights.reshape(N_SAMPLES)
  ridx = ray_indices.astype(jnp.int32)
  partials = _sc_segment_sum(ms_lin, w, ridx)
  return _tc_combine(partials)
